# Initial kernel scaffold; baseline (speedup 1.0000x reference)
import jax
import jax.numpy as jnp
from jax.experimental import pallas as pl


def kernel(x, table, concepts):
    raise NotImplementedError("write your pallas kernel here")


# speedup vs baseline: 3.7822x; 3.7822x over previous
import functools

import jax
import jax.numpy as jnp
from jax import lax
from jax.experimental import pallas as pl
from jax.experimental.pallas import tpu as pltpu
from jax.experimental.pallas import tpu_sc as plsc

VOCAB = 100000
DIM = 64
BATCH = 4096
SEQ = 200
FLAT = BATCH * SEQ

NC = 2
NS = 16
NW = NC * NS
PER_W = FLAT // NW

SUB = 128
KSUB = 4
CH = SUB * KSUB
N_CH = PER_W // CH

_mesh = plsc.VectorSubcoreMesh(
    core_axis_name="c", subcore_axis_name="s", num_cores=NC, num_subcores=NS
)


@functools.partial(
    pl.kernel,
    out_type=jax.ShapeDtypeStruct((FLAT, DIM), jnp.float32),
    mesh=_mesh,
    scratch_types=[
        pltpu.VMEM((KSUB, SUB), jnp.int32),
        pltpu.VMEM((CH, DIM), jnp.float32),
        pltpu.SemaphoreType.DMA,
    ],
    compiler_params=pltpu.CompilerParams(use_tc_tiling_on_sc=False),
)
def _sc_gather(x_hbm, tab_hbm, out_hbm, idx_v, rows_v, sem):
    wid = lax.axis_index("s") * NC + lax.axis_index("c")
    base_row = wid * (PER_W // SUB)

    def chunk(i, carry):
        row0 = base_row + i * KSUB
        pltpu.sync_copy(x_hbm.at[pl.ds(row0, KSUB)], idx_v)
        cps = []
        for j in range(KSUB):
            cps.append(
                pltpu.async_copy(
                    tab_hbm.at[idx_v.at[j]],
                    rows_v.at[pl.ds(j * SUB, SUB)],
                    sem,
                )
            )
        for cp in cps:
            cp.wait()
        pltpu.sync_copy(rows_v, out_hbm.at[pl.ds(row0 * SUB, CH)])
        return carry

    lax.fori_loop(0, N_CH, chunk, 0)


def _mask_body(x_ref, o_ref):
    o_ref[...] = x_ref[...] != 0


_tc_mask = pl.pallas_call(
    _mask_body,
    out_shape=jax.ShapeDtypeStruct((BATCH, SEQ), jnp.bool_),
    grid=(BATCH // 512,),
    in_specs=[pl.BlockSpec((512, SEQ), lambda i: (i, 0))],
    out_specs=pl.BlockSpec((512, SEQ), lambda i: (i, 0)),
)


def kernel(x, table, concepts):
    ext = jnp.concatenate([table, concepts], axis=0)
    xf = x.reshape(FLAT // SUB, SUB)
    embeds = _sc_gather(xf, ext)
    mask = _tc_mask(x)
    return embeds.reshape(BATCH, SEQ, DIM), mask


# speedup vs baseline: 4.0582x; 1.0730x over previous
import functools

import jax
import jax.numpy as jnp
from jax import lax
from jax.experimental import pallas as pl
from jax.experimental.pallas import tpu as pltpu
from jax.experimental.pallas import tpu_sc as plsc

VOCAB = 100000
DIM = 64
BATCH = 4096
SEQ = 200
FLAT = BATCH * SEQ

NC = 2
NS = 16
NW = NC * NS
PER_W = FLAT // NW

SUB = 128
KSUB = 2
CH = SUB * KSUB
N_CH = PER_W // CH
NBUF = 4
G = N_CH // NBUF

_mesh = plsc.VectorSubcoreMesh(
    core_axis_name="c", subcore_axis_name="s", num_cores=NC, num_subcores=NS
)


@functools.partial(
    pl.kernel,
    out_type=jax.ShapeDtypeStruct((FLAT, DIM), jnp.float32),
    mesh=_mesh,
    scratch_types=[
        pltpu.VMEM((NBUF, KSUB, SUB), jnp.int32),
        pltpu.VMEM((NBUF, CH, DIM), jnp.float32),
        pltpu.SemaphoreType.DMA,
        pltpu.SemaphoreType.DMA,
        pltpu.SemaphoreType.DMA,
    ],
    compiler_params=pltpu.CompilerParams(use_tc_tiling_on_sc=False),
)
def _sc_gather(x_hbm, tab_hbm, out_hbm, idx_v, rows_v, sem_i, sem_g, sem_w):
    wid = lax.axis_index("s") * NC + lax.axis_index("c")
    base_row = wid * (PER_W // SUB)

    def idx_src(i):
        return x_hbm.at[pl.ds(base_row + i * KSUB, KSUB)]

    def out_dst(i):
        return out_hbm.at[pl.ds((base_row + i * KSUB) * SUB, CH)]

    def gather_cp(b, j):
        return pltpu.make_async_copy(
            tab_hbm.at[idx_v.at[b, j]],
            rows_v.at[b, pl.ds(j * SUB, SUB)],
            sem_g,
        )

    for b in range(NBUF):
        pltpu.async_copy(idx_src(b), idx_v.at[b], sem_i)

    def outer(g, carry):
        i0 = g * NBUF
        for b in range(NBUF):
            i = i0 + b
            pltpu.make_async_copy(idx_src(i), idx_v.at[b], sem_i).wait()

            @pl.when(g > 0)
            def _():
                pltpu.make_async_copy(rows_v.at[b], out_dst(i), sem_w).wait()

            for j in range(KSUB):
                gather_cp(b, j).start()
        for b in range(NBUF):
            i = i0 + b
            for j in range(KSUB):
                gather_cp(b, j).wait()
            pltpu.async_copy(rows_v.at[b], out_dst(i), sem_w)

            @pl.when(g < G - 1)
            def _():
                pltpu.async_copy(idx_src(i + NBUF), idx_v.at[b], sem_i)

        return carry

    lax.fori_loop(0, G, outer, 0)

    for b in range(NBUF):
        pltpu.make_async_copy(
            rows_v.at[b], out_dst((G - 1) * NBUF + b), sem_w
        ).wait()


def _mask_body(x_ref, o_ref):
    o_ref[...] = x_ref[...] != 0


_tc_mask = pl.pallas_call(
    _mask_body,
    out_shape=jax.ShapeDtypeStruct((BATCH, SEQ), jnp.bool_),
    grid=(BATCH // 512,),
    in_specs=[pl.BlockSpec((512, SEQ), lambda i: (i, 0))],
    out_specs=pl.BlockSpec((512, SEQ), lambda i: (i, 0)),
)


def kernel(x, table, concepts):
    ext = jnp.concatenate([table, concepts], axis=0)
    xf = x.reshape(FLAT // SUB, SUB)
    embeds = _sc_gather(xf, ext)
    mask = _tc_mask(x)
    return embeds.reshape(BATCH, SEQ, DIM), mask
